# baseline (speedup 1.0000x reference)
import functools

import numpy as np
import jax
import jax.numpy as jnp
from jax import lax
from jax.experimental import pallas as pl
from jax.experimental.pallas import tpu as pltpu
from jax.experimental.pallas import tpu_sc as plsc

_EMB = 64
_EP = _EMB + 1
_LANES = 16
_SCALE = float(np.sqrt(np.float32(_EMB)))
_NBUF = 4
_AHEAD = 2


def _pos_encoding(length, depth):
    half = depth / 2
    positions = np.arange(length)[:, np.newaxis]
    depths = np.arange(half)[np.newaxis, :] / half
    angle_rates = 1 / 10000 ** depths
    angle_rads = positions * angle_rates
    return np.concatenate(
        [np.sin(angle_rads), np.cos(angle_rads)], axis=-1
    ).astype(np.float32)


@jax.jit
def _run(x2, tableP, pos):
    PP, NWB, S, LL = x2.shape
    L = PP * S

    mesh = plsc.VectorSubcoreMesh(core_axis_name="c", subcore_axis_name="s")
    NC = plsc.get_sparse_core_info().num_cores

    @functools.partial(
        pl.kernel,
        out_type=jax.ShapeDtypeStruct((L, _EMB // 8, NWB, 8, LL), jnp.float32),
        mesh=mesh,
        scratch_types=[
            pltpu.VMEM((PP, S, LL), jnp.int32),
            pltpu.VMEM((L, _EMB), jnp.float32),
        ]
        + [pltpu.VMEM((LL, _EP), jnp.float32) for _ in range(_NBUF)]
        + [pltpu.VMEM((_EMB, LL), jnp.float32) for _ in range(_NBUF)]
        + [
            pltpu.SemaphoreType.DMA((_NBUF,)),
            pltpu.SemaphoreType.DMA((_NBUF,)),
        ],
        compiler_params=pltpu.CompilerParams(
            use_tc_tiling_on_sc=False, needs_layout_passes=False),
    )
    def body(x2_hbm, table_hbm, pos_hbm, out_hbm, idxs, pos_v,
             r0, r1, r2, r3, t0, t1, t2, t3, gsem, osem):
        rows = [r0, r1, r2, r3]
        ts = [t0, t1, t2, t3]
        w = lax.axis_index("s") * NC + lax.axis_index("c")
        pltpu.sync_copy(pos_hbm, pos_v)
        pltpu.sync_copy(x2_hbm.at[:, w], idxs)
        iota = lax.iota(jnp.int32, _LANES)
        toks = [iota + _LANES * lc for lc in range(LL // _LANES)]

        def start_gather(p, b):
            pltpu.async_copy(
                table_hbm.at[idxs.at[p // S, p % S]], rows[b], gsem.at[b])

        def wait_gather(b):
            pltpu.make_async_copy(
                table_hbm.at[idxs.at[0, 0]], rows[b], gsem.at[b]).wait()

        def start_out(p, b):
            for g in range(_EMB // 8):
                pltpu.async_copy(
                    ts[b].at[pl.ds(g * 8, 8), :],
                    out_hbm.at[p, g, w], osem.at[b])

        def wait_out(p, b):
            for g in range(_EMB // 8):
                pltpu.make_async_copy(
                    ts[b].at[pl.ds(g * 8, 8), :],
                    out_hbm.at[p, g, w], osem.at[b]).wait()

        for b in range(_AHEAD):
            start_gather(b, b)

        def group(k, carry):
            for b in range(_NBUF):
                p = k * _NBUF + b
                wait_gather(b)

                @pl.when(p >= _NBUF)
                def _():
                    wait_out(p - _NBUF, b)

                def cc_body(cc, carry2):
                    pvv = pos_v[p, pl.ds(_LANES * cc, _LANES)]
                    for j in range(_LANES):
                        c = _LANES * cc + j
                        pv = pvv[j]
                        cols = jnp.full((_LANES,), c, jnp.int32)
                        for lc in range(LL // _LANES):
                            v16 = plsc.load_gather(rows[b], [toks[lc], cols])
                            ts[b][c, pl.ds(_LANES * lc, _LANES)] = (
                                v16 * _SCALE + pv)
                    return carry2

                lax.fori_loop(0, _EMB // _LANES, cc_body, 0, unroll=False)
                start_out(p, b)

                @pl.when(p + _AHEAD < L)
                def _():
                    start_gather(p + _AHEAD, (b + _AHEAD) % _NBUF)
            return carry

        lax.fori_loop(0, L // _NBUF, group, 0, unroll=False)
        for b in range(_NBUF):
            wait_out(L - _NBUF + b, b)

    return body(x2, tableP, pos)


def kernel(x, table):
    B, L = x.shape
    pos = jnp.asarray(_pos_encoding(L, _EMB))
    x2 = jnp.transpose(
        x.astype(jnp.int32).reshape(32, 128, L // 8, 8), (2, 0, 3, 1))
    tableP = jnp.pad(table, ((0, 0), (0, _EP - _EMB)))
    y = _run(x2, tableP, pos)
    return jnp.transpose(y, (2, 4, 0, 1, 3)).reshape(B, L, _EMB)


# speedup vs baseline: 1.2982x; 1.2982x over previous
import functools

import numpy as np
import jax
import jax.numpy as jnp
from jax import lax
from jax.experimental import pallas as pl
from jax.experimental.pallas import tpu as pltpu
from jax.experimental.pallas import tpu_sc as plsc

_EMB = 64
_LANES = 16
_SCALE = float(np.sqrt(np.float32(_EMB)))
_NBUF = 4
_NOUT = 2
_AHEAD = 2


def _pos_encoding(length, depth):
    half = depth / 2
    positions = np.arange(length)[:, np.newaxis]
    depths = np.arange(half)[np.newaxis, :] / half
    angle_rates = 1 / 10000 ** depths
    angle_rads = positions * angle_rates
    return np.concatenate(
        [np.sin(angle_rads), np.cos(angle_rads)], axis=-1
    ).astype(np.float32)


@jax.jit
def _run(x2, table, pos):
    PP, NWB, S, LL = x2.shape
    L = PP * S

    mesh = plsc.VectorSubcoreMesh(core_axis_name="c", subcore_axis_name="s")
    NC = plsc.get_sparse_core_info().num_cores

    @functools.partial(
        pl.kernel,
        out_type=jax.ShapeDtypeStruct((L, _EMB // 8, NWB, 8, LL), jnp.float32),
        mesh=mesh,
        scratch_types=[
            pltpu.VMEM((PP, S, LL), jnp.int32),
            pltpu.VMEM((L, _EMB), jnp.float32),
        ]
        + [pltpu.VMEM((LL, _EMB), jnp.float32) for _ in range(_NBUF)]
        + [pltpu.VMEM((_EMB, LL + 1), jnp.float32) for _ in range(_NBUF)]
        + [pltpu.VMEM((_EMB, LL), jnp.float32) for _ in range(_NOUT)]
        + [
            pltpu.SemaphoreType.DMA((_NBUF,)),
            pltpu.SemaphoreType.DMA((_NOUT,)),
        ],
        compiler_params=pltpu.CompilerParams(
            use_tc_tiling_on_sc=False, needs_layout_passes=False),
    )
    def body(x2_hbm, table_hbm, pos_hbm, out_hbm, idxs, pos_v,
             r0, r1, r2, r3, t0, t1, t2, t3, u0, u1, gsem, osem):
        rows = [r0, r1, r2, r3]
        ts = [t0, t1, t2, t3]
        us = [u0, u1]
        w = lax.axis_index("s") * NC + lax.axis_index("c")
        pltpu.sync_copy(pos_hbm, pos_v)
        pltpu.sync_copy(x2_hbm.at[:, w], idxs)
        iota = lax.iota(jnp.int32, _LANES)
        cvecs = [iota + _LANES * cc for cc in range(_EMB // _LANES)]

        def start_gather(p, b):
            pltpu.async_copy(
                table_hbm.at[idxs.at[p // S, p % S]], rows[b], gsem.at[b])

        def wait_gather(b):
            pltpu.make_async_copy(
                table_hbm.at[idxs.at[0, 0]], rows[b], gsem.at[b]).wait()

        def start_out(p, u):
            for g in range(_EMB // 8):
                pltpu.async_copy(
                    us[u].at[pl.ds(g * 8, 8), :],
                    out_hbm.at[p, g, w], osem.at[u])

        def wait_out(p, u):
            for g in range(_EMB // 8):
                pltpu.make_async_copy(
                    us[u].at[pl.ds(g * 8, 8), :],
                    out_hbm.at[p, g, w], osem.at[u]).wait()

        for b in range(_AHEAD):
            start_gather(b, b)

        def group(k, carry):
            for b in range(_NBUF):
                p = k * _NBUF + b
                u = b % _NOUT
                wait_gather(b)

                pvs = [pos_v[p, pl.ds(_LANES * cc, _LANES)]
                       for cc in range(_EMB // _LANES)]

                def l_body(l, carry2):
                    lsplat = jnp.full((_LANES,), l, jnp.int32)
                    for cc in range(_EMB // _LANES):
                        v = (rows[b][l, pl.ds(_LANES * cc, _LANES)] * _SCALE
                             + pvs[cc])
                        plsc.store_scatter(ts[b], [cvecs[cc], lsplat], v)
                    return carry2

                lax.fori_loop(0, LL, l_body, 0, unroll=8)

                @pl.when(p >= _NOUT)
                def _():
                    wait_out(p - _NOUT, u)

                def c_body(c, carry2):
                    for lc in range(LL // _LANES):
                        sl = pl.ds(_LANES * lc, _LANES)
                        us[u][c, sl] = ts[b][c, sl]
                    return carry2

                lax.fori_loop(0, _EMB, c_body, 0, unroll=8)
                start_out(p, u)

                @pl.when(p + _AHEAD < L)
                def _():
                    start_gather(p + _AHEAD, (b + _AHEAD) % _NBUF)
            return carry

        lax.fori_loop(0, L // _NBUF, group, 0, unroll=False)
        for t in range(_NOUT):
            wait_out(L - _NOUT + t, (L - _NOUT + t) % _NOUT)

    return body(x2, table, pos)


def kernel(x, table):
    B, L = x.shape
    pos = jnp.asarray(_pos_encoding(L, _EMB))
    x2 = jnp.transpose(
        x.astype(jnp.int32).reshape(32, 128, L // 8, 8), (2, 0, 3, 1))
    y = _run(x2, table, pos)
    return jnp.transpose(y, (2, 4, 0, 1, 3)).reshape(B, L, _EMB)


# speedup vs baseline: 1.6590x; 1.2780x over previous
import functools

import numpy as np
import jax
import jax.numpy as jnp
from jax import lax
from jax.experimental import pallas as pl
from jax.experimental.pallas import tpu as pltpu
from jax.experimental.pallas import tpu_sc as plsc

_EMB = 64
_LANES = 16
_SCALE = float(np.sqrt(np.float32(_EMB)))
_NBUF = 4
_AHEAD = 2


def _pos_encoding(length, depth):
    half = depth / 2
    positions = np.arange(length)[:, np.newaxis]
    depths = np.arange(half)[np.newaxis, :] / half
    angle_rates = 1 / 10000 ** depths
    angle_rads = positions * angle_rates
    return np.concatenate(
        [np.sin(angle_rads), np.cos(angle_rads)], axis=-1
    ).astype(np.float32)


@jax.jit
def _run(x2, table, pos):
    PP, NWB, S, LL = x2.shape
    L = PP * S

    mesh = plsc.VectorSubcoreMesh(core_axis_name="c", subcore_axis_name="s")
    NC = plsc.get_sparse_core_info().num_cores

    @functools.partial(
        pl.kernel,
        out_type=jax.ShapeDtypeStruct((L, _EMB // 8, NWB, 8, LL), jnp.float32),
        mesh=mesh,
        scratch_types=[
            pltpu.VMEM((PP, S, LL), jnp.int32),
            pltpu.VMEM((L, _EMB), jnp.float32),
        ]
        + [pltpu.VMEM((LL, _EMB), jnp.float32) for _ in range(_NBUF)]
        + [pltpu.VMEM((_EMB, LL + 1), jnp.float32) for _ in range(_NBUF)]
        + [
            pltpu.SemaphoreType.DMA((_NBUF,)),
            pltpu.SemaphoreType.DMA((_NBUF,)),
        ],
        compiler_params=pltpu.CompilerParams(
            use_tc_tiling_on_sc=False, needs_layout_passes=False),
    )
    def body(x2_hbm, table_hbm, pos_hbm, out_hbm, idxs, pos_v,
             r0, r1, r2, r3, t0, t1, t2, t3, gsem, osem):
        rows = [r0, r1, r2, r3]
        ts = [t0, t1, t2, t3]
        w = lax.axis_index("s") * NC + lax.axis_index("c")
        pltpu.sync_copy(pos_hbm, pos_v)
        pltpu.sync_copy(x2_hbm.at[:, w], idxs)
        iota = lax.iota(jnp.int32, _LANES)
        cvecs = [iota + _LANES * cc for cc in range(_EMB // _LANES)]

        def start_gather(p, b):
            pltpu.async_copy(
                table_hbm.at[idxs.at[p // S, p % S]], rows[b], gsem.at[b])

        def wait_gather(b):
            pltpu.make_async_copy(
                table_hbm.at[idxs.at[0, 0]], rows[b], gsem.at[b]).wait()

        def start_out(p, b):
            for g in range(_EMB // 8):
                pltpu.async_copy(
                    ts[b].at[pl.ds(g * 8, 8), pl.ds(0, LL)],
                    out_hbm.at[p, g, w], osem.at[b])

        def wait_out(p, b):
            for g in range(_EMB // 8):
                pltpu.make_async_copy(
                    ts[b].at[pl.ds(g * 8, 8), pl.ds(0, LL)],
                    out_hbm.at[p, g, w], osem.at[b]).wait()

        for b in range(_AHEAD):
            start_gather(b, b)

        def group(k, carry):
            for b in range(_NBUF):
                p = k * _NBUF + b
                wait_gather(b)

                @pl.when(p >= _NBUF)
                def _():
                    wait_out(p - _NBUF, b)

                pvs = [pos_v[p, pl.ds(_LANES * cc, _LANES)]
                       for cc in range(_EMB // _LANES)]

                def l_body(l, carry2):
                    lsplat = jnp.full((_LANES,), l, jnp.int32)
                    for cc in range(_EMB // _LANES):
                        v = (rows[b][l, pl.ds(_LANES * cc, _LANES)] * _SCALE
                             + pvs[cc])
                        plsc.store_scatter(ts[b], [cvecs[cc], lsplat], v)
                    return carry2

                lax.fori_loop(0, LL, l_body, 0, unroll=8)
                start_out(p, b)

                @pl.when(p + _AHEAD < L)
                def _():
                    start_gather(p + _AHEAD, (b + _AHEAD) % _NBUF)
            return carry

        lax.fori_loop(0, L // _NBUF, group, 0, unroll=False)
        for b in range(_NBUF):
            wait_out(L - _NBUF + b, b)

    return body(x2, table, pos)


def kernel(x, table):
    B, L = x.shape
    pos = jnp.asarray(_pos_encoding(L, _EMB))
    x2 = jnp.transpose(
        x.astype(jnp.int32).reshape(32, 128, L // 8, 8), (2, 0, 3, 1))
    y = _run(x2, table, pos)
    return jnp.transpose(y, (2, 4, 0, 1, 3)).reshape(B, L, _EMB)
